# Initial kernel scaffold; baseline (speedup 1.0000x reference)
import functools

import jax
import jax.numpy as jnp
from jax import lax
from jax.experimental import pallas as pl
from jax.experimental.pallas import tpu as pltpu
from jax.experimental.pallas import tpu_sc as plsc

N = 50000
E = 800000
F_IN = 8
H = 64
HH = 32
C_OUT = 10
G = 128

NCORE = 2
NSUB = 16
CHUNK = 128
CPB = 17
NBLK = 23
EPT = CHUNK * CPB * NBLK
E_PAD = EPT * NSUB
EROWS = E_PAD // CHUNK
RPT_E = CPB * NBLK

N_ACC = 50048
RPT = N_ACC // NSUB
ZROWS = 391

DEG_PAD = 51200
DPT = DEG_PAD // NSUB

BN = 6250

_MESH = plsc.VectorSubcoreMesh(
    core_axis_name="c", subcore_axis_name="s",
    num_cores=NCORE, num_subcores=NSUB,
)

_Z16 = jnp.zeros((16,), jnp.float32)


@functools.partial(
    pl.kernel,
    out_type=jax.ShapeDtypeStruct((DEG_PAD,), jnp.float32),
    mesh=_MESH,
    scratch_types=[
        pltpu.VMEM_SHARED((DEG_PAD,), jnp.float32),
        pltpu.VMEM((CPB, CHUNK), jnp.int32),
        pltpu.VMEM((CHUNK,), jnp.float32),
        pltpu.VMEM((DPT,), jnp.float32),
    ],
)
def _deg_kernel(dstp, out, acc, dbuf, ones, zbuf):
    c = lax.axis_index("c")
    s = lax.axis_index("s")

    @pl.when(c == 0)
    def _():
        def zfill(i, carry):
            zbuf[pl.ds(i * 16, 16)] = _Z16
            return carry
        lax.fori_loop(0, DPT // 16, zfill, 0)
        pltpu.sync_copy(zbuf, acc.at[pl.ds(s * DPT, DPT)])

        def ofill(i, carry):
            ones[pl.ds(i * 16, 16)] = _Z16 + 1.0
            return carry
        lax.fori_loop(0, CHUNK // 16, ofill, 0)
        plsc.subcore_barrier()

        rb0 = s * RPT_E

        def block(b, carry):
            pltpu.sync_copy(dstp.at[pl.ds(rb0 + b * CPB, CPB)], dbuf)
            for j in range(CPB):
                pltpu.sync_copy(ones, acc.at[dbuf.at[j]], add=True)
            return carry

        lax.fori_loop(0, NBLK, block, 0)
        plsc.subcore_barrier()
        pltpu.sync_copy(acc.at[pl.ds(s * DPT, DPT)], out.at[pl.ds(s * DPT, DPT)])


@functools.partial(
    pl.kernel,
    out_type=jax.ShapeDtypeStruct((NCORE, N_ACC, HH), jnp.float32),
    mesh=_MESH,
    scratch_types=[
        pltpu.VMEM_SHARED((N_ACC, HH), jnp.float32),
        pltpu.VMEM((CPB, CHUNK), jnp.int32),
        pltpu.VMEM((CPB, CHUNK), jnp.int32),
        pltpu.VMEM((CPB, CHUNK), jnp.int32),
        pltpu.VMEM((CPB * CHUNK, HH), jnp.float32),
        pltpu.VMEM((ZROWS, HH), jnp.float32),
        pltpu.SemaphoreType.DMA,
    ],
)
def _scatter_kernel(g2, srcp, dstp, out, acc, sbuf, gbuf, dbuf, rows, zbuf, sem):
    c = lax.axis_index("c")
    s = lax.axis_index("s")

    def zrow(r, carry):
        zbuf[r, pl.ds(0, 16)] = _Z16
        zbuf[r, pl.ds(16, 16)] = _Z16
        return carry
    lax.fori_loop(0, ZROWS, zrow, 0)
    row0 = s * RPT
    for jz in range(RPT // ZROWS):
        pltpu.sync_copy(zbuf, acc.at[pl.ds(row0 + jz * ZROWS, ZROWS)])
    plsc.subcore_barrier()

    rb0 = s * RPT_E

    def block(b, carry):
        rb = rb0 + b * CPB
        pltpu.sync_copy(srcp.at[pl.ds(rb, CPB)], sbuf)
        pltpu.sync_copy(dstp.at[pl.ds(rb, CPB)], dbuf)

        def xf(i, carry2):
            r = i // 8
            k = (i % 8) * 16
            v = sbuf[r, pl.ds(k, 16)]
            gbuf[r, pl.ds(k, 16)] = v + v + c
            return carry2
        lax.fori_loop(0, CPB * 8, xf, 0)

        handles = []
        for j in range(CPB):
            handles.append(pltpu.async_copy(
                g2.at[gbuf.at[j]], rows.at[pl.ds(j * CHUNK, CHUNK)], sem))
        for h in handles:
            h.wait()
        for j in range(CPB):
            pltpu.sync_copy(rows.at[pl.ds(j * CHUNK, CHUNK)],
                            acc.at[dbuf.at[j]], add=True)
        return carry

    lax.fori_loop(0, NBLK, block, 0)
    plsc.subcore_barrier()
    pltpu.sync_copy(acc.at[pl.ds(row0, RPT)], out.at[c, pl.ds(row0, RPT)])


def _mm1_body(x_ref, dg_ref, w_ref, o_ref):
    d = lax.rsqrt(dg_ref[...])
    o_ref[...] = jnp.dot(x_ref[...] * d, w_ref[...],
                         preferred_element_type=jnp.float32)


_mm1 = pl.pallas_call(
    _mm1_body,
    grid=(N // BN,),
    in_specs=[
        pl.BlockSpec((BN, F_IN), lambda i: (i, 0)),
        pl.BlockSpec((BN, 1), lambda i: (i, 0)),
        pl.BlockSpec((F_IN, H), lambda i: (0, 0)),
    ],
    out_specs=pl.BlockSpec((BN, H), lambda i: (i, 0)),
    out_shape=jax.ShapeDtypeStruct((N, H), jnp.float32),
)


def _mml_body(s_ref, g_ref, dg_ref, b_ref, w_ref, o_ref):
    d = lax.rsqrt(dg_ref[...])
    sb = jnp.concatenate([s_ref[0], s_ref[1]], axis=1)
    h = jnp.maximum(d * (sb + g_ref[...]) + b_ref[...], 0.0)
    o_ref[...] = jnp.dot(h * d, w_ref[...], preferred_element_type=jnp.float32)


_mml = pl.pallas_call(
    _mml_body,
    grid=(N // BN,),
    in_specs=[
        pl.BlockSpec((NCORE, BN, HH), lambda i: (0, i, 0)),
        pl.BlockSpec((BN, H), lambda i: (i, 0)),
        pl.BlockSpec((BN, 1), lambda i: (i, 0)),
        pl.BlockSpec((1, H), lambda i: (0, 0)),
        pl.BlockSpec((H, H), lambda i: (0, 0)),
    ],
    out_specs=pl.BlockSpec((BN, H), lambda i: (i, 0)),
    out_shape=jax.ShapeDtypeStruct((N, H), jnp.float32),
)


def _pool_body(s_ref, g_ref, dg_ref, b_ref, batch_ref, p_ref):
    i = pl.program_id(0)
    d = lax.rsqrt(dg_ref[...])
    sb = jnp.concatenate([s_ref[0], s_ref[1]], axis=1)
    h = jnp.maximum(d * (sb + g_ref[...]) + b_ref[...], 0.0)
    hh = jnp.concatenate([h, jnp.ones((BN, 1), jnp.float32)], axis=1)
    oh = (batch_ref[...] ==
          lax.broadcasted_iota(jnp.int32, (BN, G), 1)).astype(jnp.float32)
    contrib = lax.dot_general(oh, hh, (((0,), (0,)), ((), ())),
                              preferred_element_type=jnp.float32)

    @pl.when(i == 0)
    def _():
        p_ref[...] = contrib

    @pl.when(i != 0)
    def _():
        p_ref[...] = p_ref[...] + contrib


_pool = pl.pallas_call(
    _pool_body,
    grid=(N // BN,),
    in_specs=[
        pl.BlockSpec((NCORE, BN, HH), lambda i: (0, i, 0)),
        pl.BlockSpec((BN, H), lambda i: (i, 0)),
        pl.BlockSpec((BN, 1), lambda i: (i, 0)),
        pl.BlockSpec((1, H), lambda i: (0, 0)),
        pl.BlockSpec((BN, 1), lambda i: (i, 0)),
    ],
    out_specs=pl.BlockSpec((G, H + 1), lambda i: (0, 0)),
    out_shape=jax.ShapeDtypeStruct((G, H + 1), jnp.float32),
)


def _head_body(p_ref, w_ref, b_ref, o_ref):
    P = p_ref[...]
    cnt = jnp.maximum(P[:, H:H + 1], 1.0)
    pooled = P[:, :H] / cnt
    logits = jnp.dot(pooled, w_ref[...],
                     preferred_element_type=jnp.float32) + b_ref[...]
    m = jnp.max(logits, axis=1, keepdims=True)
    e = jnp.exp(logits - m)
    lse = jnp.log(jnp.sum(e, axis=1, keepdims=True)) + m
    o_ref[...] = logits - lse


_head = pl.pallas_call(
    _head_body,
    out_shape=jax.ShapeDtypeStruct((G, C_OUT), jnp.float32),
)


def kernel(x, edge_index, batch, W1, b1, W2, b2, W3, b3, W4, b4, W5, b5,
           Wout, bout):
    src = edge_index[0]
    dst = edge_index[1]
    pad = E_PAD - E
    srcp = jnp.concatenate(
        [src, jnp.zeros((pad,), jnp.int32)]).reshape(EROWS, CHUNK)
    dstp = jnp.concatenate(
        [dst, jnp.full((pad,), N, jnp.int32)]).reshape(EROWS, CHUNK)

    degp = _deg_kernel(dstp)
    degsum = (1.0 + degp[:N]).reshape(N, 1)
    batch2 = batch.reshape(N, 1)

    g = _mm1(x, degsum, W1)
    for (W, b) in ((W2, b1), (W3, b2), (W4, b3), (W5, b4)):
        S = _scatter_kernel(g.reshape(2 * N, HH), srcp, dstp)
        g = _mml(S, g, degsum, b.reshape(1, H), W)
    S = _scatter_kernel(g.reshape(2 * N, HH), srcp, dstp)
    P = _pool(S, g, degsum, b5.reshape(1, H), batch2)
    return _head(P, Wout, bout.reshape(1, C_OUT))


# speedup vs baseline: 18.8842x; 18.8842x over previous
import functools

import jax
import jax.numpy as jnp
from jax import lax
from jax.experimental import pallas as pl
from jax.experimental.pallas import tpu as pltpu
from jax.experimental.pallas import tpu_sc as plsc

N = 50000
E = 800000
F_IN = 8
H = 64
HH = 32
C_OUT = 10
G = 128

NCORE = 2
NSUB = 16
CHUNK = 128
CPB = 8
NBLK = 49
EPT = CHUNK * CPB * NBLK
E_PAD = EPT * NSUB
EROWS = E_PAD // CHUNK
RPT_E = CPB * NBLK

N_ACC = 50048
RPT = N_ACC // NSUB
ZROWS = 136
WAVE = 4

DEG_PAD = 51200
DPT = DEG_PAD // NSUB

BN = 5000

_MESH = plsc.VectorSubcoreMesh(
    core_axis_name="c", subcore_axis_name="s",
    num_cores=NCORE, num_subcores=NSUB,
)

def _z16():
    return jnp.zeros((16,), jnp.float32)


@functools.partial(
    pl.kernel,
    out_type=jax.ShapeDtypeStruct((DEG_PAD,), jnp.float32),
    mesh=_MESH,
    compiler_params=pltpu.CompilerParams(use_tc_tiling_on_sc=False),
    scratch_types=[
        pltpu.VMEM_SHARED((DEG_PAD,), jnp.float32),
        pltpu.VMEM((CPB, CHUNK), jnp.int32),
        pltpu.VMEM((CHUNK,), jnp.float32),
        pltpu.VMEM((DPT,), jnp.float32),
    ],
)
def _deg_kernel(dstp, out, acc, dbuf, ones, zbuf):
    c = lax.axis_index("c")
    s = lax.axis_index("s")

    @pl.when(c == 0)
    def _():
        def zfill(i, carry):
            zbuf[pl.ds(i * 16, 16)] = _z16()
            return carry
        lax.fori_loop(0, DPT // 16, zfill, 0)
        pltpu.sync_copy(zbuf, acc.at[pl.ds(s * DPT, DPT)])

        def ofill(i, carry):
            ones[pl.ds(i * 16, 16)] = _z16() + 1.0
            return carry
        lax.fori_loop(0, CHUNK // 16, ofill, 0)
        plsc.subcore_barrier()

        rb0 = s * RPT_E

        def block(b, carry):
            pltpu.sync_copy(dstp.at[pl.ds(rb0 + b * CPB, CPB)], dbuf)
            for j in range(CPB):
                pltpu.sync_copy(ones, acc.at[dbuf.at[j]], add=True)
            return carry

        lax.fori_loop(0, NBLK, block, 0)
        plsc.subcore_barrier()
        pltpu.sync_copy(acc.at[pl.ds(s * DPT, DPT)], out.at[pl.ds(s * DPT, DPT)])


@functools.partial(
    pl.kernel,
    out_type=jax.ShapeDtypeStruct((NCORE, N_ACC, HH), jnp.float32),
    mesh=_MESH,
    compiler_params=pltpu.CompilerParams(use_tc_tiling_on_sc=False),
    scratch_types=[
        pltpu.VMEM_SHARED((N_ACC, HH), jnp.float32),
        pltpu.VMEM((CPB, CHUNK), jnp.int32),
        pltpu.VMEM((CPB, CHUNK), jnp.int32),
        pltpu.VMEM((CPB, CHUNK), jnp.int32),
        pltpu.VMEM((WAVE * CHUNK, HH), jnp.float32),
        pltpu.VMEM((ZROWS, HH), jnp.float32),
        pltpu.SemaphoreType.DMA,
    ],
)
def _scatter_kernel(g2, srcp, dstp, out, acc, sbuf, gbuf, dbuf, rows, zbuf, sem):
    c = lax.axis_index("c")
    s = lax.axis_index("s")

    def zrow(r, carry):
        zbuf[r, pl.ds(0, 16)] = _z16()
        zbuf[r, pl.ds(16, 16)] = _z16()
        return carry
    lax.fori_loop(0, ZROWS, zrow, 0)
    row0 = s * RPT
    for jz in range(RPT // ZROWS):
        pltpu.sync_copy(zbuf, acc.at[pl.ds(row0 + jz * ZROWS, ZROWS)])
    plsc.subcore_barrier()

    rb0 = s * RPT_E

    def block(b, carry):
        rb = rb0 + b * CPB
        pltpu.sync_copy(srcp.at[pl.ds(rb, CPB)], sbuf)
        pltpu.sync_copy(dstp.at[pl.ds(rb, CPB)], dbuf)

        def xf(i, carry2):
            r = i // 8
            k = (i % 8) * 16
            v = sbuf[r, pl.ds(k, 16)]
            gbuf[r, pl.ds(k, 16)] = v + v + c
            return carry2
        lax.fori_loop(0, CPB * 8, xf, 0)

        for w in range(CPB // WAVE):
            handles = []
            for j in range(WAVE):
                handles.append(pltpu.async_copy(
                    g2.at[gbuf.at[w * WAVE + j]],
                    rows.at[pl.ds(j * CHUNK, CHUNK)], sem))
            for h in handles:
                h.wait()
            for j in range(WAVE):
                pltpu.sync_copy(rows.at[pl.ds(j * CHUNK, CHUNK)],
                                acc.at[dbuf.at[w * WAVE + j]], add=True)
        return carry

    lax.fori_loop(0, NBLK, block, 0)
    plsc.subcore_barrier()
    pltpu.sync_copy(acc.at[pl.ds(row0, RPT)], out.at[c, pl.ds(row0, RPT)])


def _mm1_body(x_ref, dg_ref, w_ref, o_ref):
    d = lax.rsqrt(dg_ref[...])
    o_ref[...] = jnp.dot(x_ref[...] * d, w_ref[...],
                         preferred_element_type=jnp.float32)


_mm1 = pl.pallas_call(
    _mm1_body,
    grid=(N // BN,),
    in_specs=[
        pl.BlockSpec((BN, F_IN), lambda i: (i, 0)),
        pl.BlockSpec((BN, 1), lambda i: (i, 0)),
        pl.BlockSpec((F_IN, H), lambda i: (0, 0)),
    ],
    out_specs=pl.BlockSpec((BN, H), lambda i: (i, 0)),
    out_shape=jax.ShapeDtypeStruct((N, H), jnp.float32),
)


def _mml_body(s_ref, g_ref, dg_ref, b_ref, w_ref, o_ref):
    d = lax.rsqrt(dg_ref[...])
    sb = jnp.concatenate([s_ref[0], s_ref[1]], axis=1)
    h = jnp.maximum(d * (sb + g_ref[...]) + b_ref[...], 0.0)
    o_ref[...] = jnp.dot(h * d, w_ref[...], preferred_element_type=jnp.float32)


_mml = pl.pallas_call(
    _mml_body,
    grid=(N // BN,),
    in_specs=[
        pl.BlockSpec((NCORE, BN, HH), lambda i: (0, i, 0)),
        pl.BlockSpec((BN, H), lambda i: (i, 0)),
        pl.BlockSpec((BN, 1), lambda i: (i, 0)),
        pl.BlockSpec((1, H), lambda i: (0, 0)),
        pl.BlockSpec((H, H), lambda i: (0, 0)),
    ],
    out_specs=pl.BlockSpec((BN, H), lambda i: (i, 0)),
    out_shape=jax.ShapeDtypeStruct((N, H), jnp.float32),
)


def _pool_body(s_ref, g_ref, dg_ref, b_ref, batch_ref, p_ref):
    i = pl.program_id(0)
    d = lax.rsqrt(dg_ref[...])
    sb = jnp.concatenate([s_ref[0], s_ref[1]], axis=1)
    h = jnp.maximum(d * (sb + g_ref[...]) + b_ref[...], 0.0)
    hh = jnp.concatenate([h, jnp.ones((BN, 1), jnp.float32)], axis=1)
    oh = (batch_ref[...] ==
          lax.broadcasted_iota(jnp.int32, (BN, G), 1)).astype(jnp.float32)
    contrib = lax.dot_general(oh, hh, (((0,), (0,)), ((), ())),
                              preferred_element_type=jnp.float32)

    @pl.when(i == 0)
    def _():
        p_ref[...] = contrib

    @pl.when(i != 0)
    def _():
        p_ref[...] = p_ref[...] + contrib


_pool = pl.pallas_call(
    _pool_body,
    grid=(N // BN,),
    in_specs=[
        pl.BlockSpec((NCORE, BN, HH), lambda i: (0, i, 0)),
        pl.BlockSpec((BN, H), lambda i: (i, 0)),
        pl.BlockSpec((BN, 1), lambda i: (i, 0)),
        pl.BlockSpec((1, H), lambda i: (0, 0)),
        pl.BlockSpec((BN, 1), lambda i: (i, 0)),
    ],
    out_specs=pl.BlockSpec((G, H + 1), lambda i: (0, 0)),
    out_shape=jax.ShapeDtypeStruct((G, H + 1), jnp.float32),
)


def _head_body(p_ref, w_ref, b_ref, o_ref):
    P = p_ref[...]
    cnt = jnp.maximum(P[:, H:H + 1], 1.0)
    pooled = P[:, :H] / cnt
    logits = jnp.dot(pooled, w_ref[...],
                     preferred_element_type=jnp.float32) + b_ref[...]
    m = jnp.max(logits, axis=1, keepdims=True)
    e = jnp.exp(logits - m)
    lse = jnp.log(jnp.sum(e, axis=1, keepdims=True)) + m
    o_ref[...] = logits - lse


_head = pl.pallas_call(
    _head_body,
    out_shape=jax.ShapeDtypeStruct((G, C_OUT), jnp.float32),
)


def kernel(x, edge_index, batch, W1, b1, W2, b2, W3, b3, W4, b4, W5, b5,
           Wout, bout):
    src = edge_index[0]
    dst = edge_index[1]
    pad = E_PAD - E
    srcp = jnp.concatenate(
        [src, jnp.zeros((pad,), jnp.int32)]).reshape(EROWS, CHUNK)
    dstp = jnp.concatenate(
        [dst, jnp.full((pad,), N, jnp.int32)]).reshape(EROWS, CHUNK)

    degp = _deg_kernel(dstp)
    degsum = (1.0 + degp[:N]).reshape(N, 1)
    batch2 = batch.reshape(N, 1)

    g = _mm1(x, degsum, W1)
    for (W, b) in ((W2, b1), (W3, b2), (W4, b3), (W5, b4)):
        S = _scatter_kernel(g.reshape(2 * N, HH), srcp, dstp)
        g = _mml(S, g, degsum, b.reshape(1, H), W)
    S = _scatter_kernel(g.reshape(2 * N, HH), srcp, dstp)
    P = _pool(S, g, degsum, b5.reshape(1, H), batch2)
    return _head(P, Wout, bout.reshape(1, C_OUT))
